# baseline (speedup 1.0000x reference)
import functools

import jax
import jax.numpy as jnp
from jax import lax
from jax.experimental import pallas as pl
from jax.experimental.pallas import tpu as pltpu
from jax.experimental.pallas import tpu_sc as plsc

D_MODEL = 1024
N_ROWS = 4 * 8192
NSEM = 4


def _make_gather():
    info = plsc.get_sparse_core_info()
    nc, ns = info.num_cores, info.num_subcores
    nw = nc * ns
    rows_per_w = N_ROWS // nw

    mesh = plsc.VectorSubcoreMesh(core_axis_name="c", subcore_axis_name="s")

    @functools.partial(
        pl.kernel,
        mesh=mesh,
        out_type=jax.ShapeDtypeStruct((N_ROWS, D_MODEL), jnp.float32),
        scratch_types=[
            pltpu.VMEM((rows_per_w,), jnp.int32),
            *([pltpu.SemaphoreType.DMA] * NSEM),
        ],
    )
    def gather_kernel(idx_hbm, table_hbm, out_hbm, idx_v, *sems):
        wid = lax.axis_index("s") * nc + lax.axis_index("c")
        base = wid * rows_per_w

        pltpu.sync_copy(idx_hbm.at[wid], idx_v)

        sem = sems[0]

        def body(g, _):
            vec = idx_v[pl.ds(g * 16, 16)]
            for k in range(16):
                pltpu.async_copy(table_hbm.at[pl.ds(vec[k], 1)],
                                 out_hbm.at[pl.ds(base + g * 16 + k, 1)],
                                 sem)

            @pl.when(g >= 1)
            def _():
                for k in range(16):
                    pltpu.make_async_copy(
                        table_hbm.at[pl.ds(0, 1)],
                        out_hbm.at[pl.ds(base + (g - 1) * 16 + k, 1)],
                        sem).wait()

            return 0

        n_groups = rows_per_w // 16
        lax.fori_loop(0, n_groups, body, 0)

        for k in range(16):
            pltpu.make_async_copy(
                table_hbm.at[pl.ds(0, 1)],
                out_hbm.at[pl.ds(base + (n_groups - 1) * 16 + k, 1)],
                sem).wait()

    return gather_kernel


_gather = _make_gather()


@jax.jit
def kernel(token_positions, pe):
    b, t = token_positions.shape
    info = plsc.get_sparse_core_info()
    nw = info.num_cores * info.num_subcores
    rows_per_w = N_ROWS // nw
    idx = token_positions.astype(jnp.int32).reshape(nw, rows_per_w)
    out = _gather(idx, pe)
    return out.reshape(b, t, D_MODEL)


# speedup vs baseline: 36.1803x; 36.1803x over previous
import functools

import jax
import jax.numpy as jnp
from jax import lax
from jax.experimental import pallas as pl
from jax.experimental.pallas import tpu as pltpu
from jax.experimental.pallas import tpu_sc as plsc

D_MODEL = 1024
N_ROWS = 4 * 8192
CHUNK = 8
NBUF = 8
LEAD = 4


def _make_gather():
    info = plsc.get_sparse_core_info()
    nc, ns = info.num_cores, info.num_subcores
    nw = nc * ns
    rows_per_w = N_ROWS // nw
    n_chunks = rows_per_w // CHUNK

    mesh = plsc.VectorSubcoreMesh(core_axis_name="c", subcore_axis_name="s")

    @functools.partial(
        pl.kernel,
        mesh=mesh,
        out_type=jax.ShapeDtypeStruct((N_ROWS, D_MODEL), jnp.float32),
        scratch_types=[
            pltpu.VMEM((n_chunks, CHUNK), jnp.int32),
            *([pltpu.VMEM((CHUNK, D_MODEL), jnp.float32)] * NBUF),
            *([pltpu.SemaphoreType.DMA] * NBUF),
            *([pltpu.SemaphoreType.DMA] * NBUF),
        ],
    )
    def gather_kernel(idx_hbm, table_hbm, out_hbm, idx_v, *scratch):
        bufs = scratch[:NBUF]
        gsems = scratch[NBUF:2 * NBUF]
        wsems = scratch[2 * NBUF:]

        wid = lax.axis_index("s") * nc + lax.axis_index("c")
        base = wid * rows_per_w

        pltpu.sync_copy(idx_hbm.at[wid], idx_v)

        def gather(c, b):
            pltpu.async_copy(table_hbm.at[idx_v.at[c]], bufs[b], gsems[b])

        def write(c, b):
            pltpu.async_copy(
                bufs[b], out_hbm.at[pl.ds(base + c * CHUNK, CHUNK)], wsems[b])

        for c in range(LEAD):
            gather(c, c)

        def step(c, b):
            bg = (b + LEAD) % NBUF

            @pl.when(c + LEAD < n_chunks)
            def _():
                @pl.when(c >= NBUF - LEAD)
                def _():
                    pltpu.make_async_copy(
                        bufs[bg],
                        out_hbm.at[
                            pl.ds(base + (c - (NBUF - LEAD)) * CHUNK, CHUNK)],
                        wsems[bg]).wait()
                gather(c + LEAD, bg)

            pltpu.make_async_copy(
                table_hbm.at[idx_v.at[c]], bufs[b], gsems[b]).wait()
            write(c, b)

        def body(i, _):
            c = NBUF * i
            for b in range(NBUF):
                step(c + b, b)
            return 0

        lax.fori_loop(0, n_chunks // NBUF, body, 0)

        for b in range(NBUF):
            c = n_chunks - NBUF + b
            pltpu.make_async_copy(
                bufs[b], out_hbm.at[pl.ds(base + c * CHUNK, CHUNK)],
                wsems[b]).wait()

    return gather_kernel


_gather = _make_gather()


@jax.jit
def kernel(token_positions, pe):
    b, t = token_positions.shape
    info = plsc.get_sparse_core_info()
    nw = info.num_cores * info.num_subcores
    rows_per_w = N_ROWS // nw
    idx = token_positions.astype(jnp.int32).reshape(nw, rows_per_w // CHUNK,
                                                    CHUNK)
    out = _gather(idx, pe)
    return out.reshape(b, t, D_MODEL)


# speedup vs baseline: 36.3741x; 1.0054x over previous
import functools

import jax
import jax.numpy as jnp
from jax import lax
from jax.experimental import pallas as pl
from jax.experimental.pallas import tpu as pltpu
from jax.experimental.pallas import tpu_sc as plsc

D_MODEL = 1024
N_ROWS = 4 * 8192
CHUNK = 16
NBUF = 4


def _make_gather():
    info = plsc.get_sparse_core_info()
    nc, ns = info.num_cores, info.num_subcores
    nw = nc * ns
    rows_per_w = N_ROWS // nw
    n_chunks = rows_per_w // CHUNK

    mesh = plsc.VectorSubcoreMesh(core_axis_name="c", subcore_axis_name="s")

    @functools.partial(
        pl.kernel,
        mesh=mesh,
        out_type=jax.ShapeDtypeStruct((N_ROWS, D_MODEL), jnp.float32),
        scratch_types=[
            pltpu.VMEM((n_chunks, CHUNK), jnp.int32),
            *([pltpu.VMEM((CHUNK, D_MODEL), jnp.float32)] * NBUF),
            *([pltpu.SemaphoreType.DMA] * NBUF),
            *([pltpu.SemaphoreType.DMA] * NBUF),
        ],
    )
    def gather_kernel(idx_hbm, table_hbm, out_hbm, idx_v, *scratch):
        bufs = scratch[:NBUF]
        gsems = scratch[NBUF:2 * NBUF]
        wsems = scratch[2 * NBUF:]

        wid = lax.axis_index("s") * nc + lax.axis_index("c")
        base = wid * rows_per_w

        pltpu.sync_copy(idx_hbm.at[wid], idx_v)

        def gather(c, b):
            pltpu.async_copy(table_hbm.at[idx_v.at[c]], bufs[b], gsems[b])

        def write(c, b):
            pltpu.async_copy(
                bufs[b], out_hbm.at[pl.ds(base + c * CHUNK, CHUNK)], wsems[b])

        gather(0, 0)
        gather(1, 1)

        def step(c, b):
            b2 = (b + 2) % NBUF

            @pl.when(c + 2 < n_chunks)
            def _():
                @pl.when(c >= 2)
                def _():
                    pltpu.make_async_copy(
                        bufs[b2],
                        out_hbm.at[pl.ds(base + (c - 2) * CHUNK, CHUNK)],
                        wsems[b2]).wait()
                gather(c + 2, b2)

            pltpu.make_async_copy(
                table_hbm.at[idx_v.at[c]], bufs[b], gsems[b]).wait()
            write(c, b)

        def body(i, _):
            c = NBUF * i
            for b in range(NBUF):
                step(c + b, b)
            return 0

        lax.fori_loop(0, n_chunks // NBUF, body, 0)

        for b in range(NBUF):
            c = n_chunks - NBUF + b
            pltpu.make_async_copy(
                bufs[b], out_hbm.at[pl.ds(base + c * CHUNK, CHUNK)],
                wsems[b]).wait()

    return gather_kernel


_gather = _make_gather()


@jax.jit
def kernel(token_positions, pe):
    b, t = token_positions.shape
    info = plsc.get_sparse_core_info()
    nw = info.num_cores * info.num_subcores
    rows_per_w = N_ROWS // nw
    idx = token_positions.astype(jnp.int32).reshape(nw, rows_per_w // CHUNK,
                                                    CHUNK)
    out = _gather(idx, pe)
    return out.reshape(b, t, D_MODEL)
